# baseline (speedup 1.0000x reference)
import dataclasses

import jax
import jax.numpy as jnp
from jax import lax
from jax.experimental import pallas as pl
from jax.experimental.pallas import tpu as pltpu
from jax.experimental.pallas import tpu_sc as plsc

N = 10000
D = 256
DW = D // 2
S = 25

NUM_WORKERS = 32
NODES_PER_WORKER = 320
N_PAD = NUM_WORKERS * NODES_PER_WORKER
GROUP = 8
GROUPS_PER_WORKER = NODES_PER_WORKER // GROUP
IDX_PER_GROUP = GROUP * S
HALF0 = 104
HALF1 = IDX_PER_GROUP - HALF0
LANES = 16
WCHUNKS = DW // LANES


def _sc_body(h_hbm, idx_hbm, out_hbm, idx_all, rows0, rows1, out0, out1,
             semr0, semr1, semo0, semo1):
  core = lax.axis_index("c")
  sub = lax.axis_index("s")
  wid = sub * 2 + core

  idx_base = wid * (NODES_PER_WORKER * S)
  node_base = wid * NODES_PER_WORKER

  rows = (rows0, rows1)
  outs = (out0, out1)
  semr = (semr0, semr1)
  semo = (semo0, semo1)

  pltpu.sync_copy(idx_hbm.at[pl.ds(idx_base, NODES_PER_WORKER * S)], idx_all)

  def issue_gather(g, b):
    off = g * IDX_PER_GROUP
    pltpu.async_copy(h_hbm.at[idx_all.at[pl.ds(off, HALF0)]],
                     rows[b].at[pl.ds(0, HALF0)], semr[b])
    pltpu.async_copy(h_hbm.at[idx_all.at[pl.ds(off + HALF0, HALF1)]],
                     rows[b].at[pl.ds(HALF0, HALF1)], semr[b])

  def wait_gather(b):
    pltpu.make_async_copy(h_hbm.at[pl.ds(0, IDX_PER_GROUP)], rows[b],
                          semr[b]).wait()

  def wait_store(b):
    pltpu.make_async_copy(outs[b], out_hbm.at[pl.ds(0, GROUP)],
                          semo[b]).wait()

  issue_gather(0, 0)

  @pl.loop(0, GROUPS_PER_WORKER, step=2)
  def _(g):
    for b in range(2):
      gg = g + b
      nxt = gg + 1

      @pl.when(nxt < GROUPS_PER_WORKER)
      def _():
        issue_gather(nxt, 1 - b)

      wait_gather(b)

      @pl.when(gg >= 2)
      def _():
        wait_store(b)

      for n in range(GROUP):
        def acc_body(r, accs, n=n):
          row = n * S + r
          new = []
          for c in range(WCHUNKS):
            w = rows[b][row, pl.ds(c * LANES, LANES)]
            v = plsc.bitcast(w, jnp.bfloat16)
            lo, hi = plsc.unpack(v, format=plsc.PackFormat.INTERLEAVED)
            new.append(accs[2 * c] + lo)
            new.append(accs[2 * c + 1] + hi)
          return tuple(new)
        accs = lax.fori_loop(
            0, S, acc_body,
            tuple(jnp.zeros((LANES,), jnp.float32)
                  for _ in range(2 * WCHUNKS)),
            unroll=5)
        for c in range(WCHUNKS):
          packed = plsc.pack(accs[2 * c] * (1.0 / S),
                             accs[2 * c + 1] * (1.0 / S),
                             format=plsc.PackFormat.INTERLEAVED)
          outs[b][n, pl.ds(c * LANES, LANES)] = plsc.bitcast(
              packed, jnp.int32)

      pltpu.async_copy(outs[b],
                       out_hbm.at[pl.ds(node_base + gg * GROUP, GROUP)],
                       semo[b])

  wait_store(0)
  wait_store(1)


@jax.jit
def _sc_gather_mean(h_i32, flat_idx):
  mesh = plsc.VectorSubcoreMesh(core_axis_name="c", subcore_axis_name="s")
  cp = pltpu.CompilerParams()
  if "needs_layout_passes" in pltpu.CompilerParams.__dataclass_fields__:
    cp = dataclasses.replace(cp, needs_layout_passes=False)
  kern = pl.kernel(
      _sc_body,
      compiler_params=cp,
      out_type=jax.ShapeDtypeStruct((N_PAD, DW), jnp.int32),
      mesh=mesh,
      scratch_types=[
          pltpu.VMEM((NODES_PER_WORKER * S,), jnp.int32),
          pltpu.VMEM((IDX_PER_GROUP, DW), jnp.int32),
          pltpu.VMEM((IDX_PER_GROUP, DW), jnp.int32),
          pltpu.VMEM((GROUP, DW), jnp.int32),
          pltpu.VMEM((GROUP, DW), jnp.int32),
          pltpu.SemaphoreType.DMA,
          pltpu.SemaphoreType.DMA,
          pltpu.SemaphoreType.DMA,
          pltpu.SemaphoreType.DMA,
      ],
  )
  return kern(h_i32, flat_idx)


def _pack_bf16(x_bf16):
  return lax.bitcast_convert_type(
      x_bf16.reshape(N, DW, 2), jnp.int32)


def _unpack_bf16(x_i32):
  return lax.bitcast_convert_type(x_i32, jnp.bfloat16).reshape(N_PAD, D)


def _dense_bn_body(h_ref, agg_ref, wt_ref, wb_ref, b_ref, g_ref, be_ref,
                   o_ref, ob_ref):
  x = jnp.dot(h_ref[...], wt_ref[...], preferred_element_type=jnp.float32)
  x = x + jnp.dot(agg_ref[...], wb_ref[...],
                  preferred_element_type=jnp.float32)
  x = x + b_ref[...]
  x = jnp.maximum(x, 0.0)
  mu = jnp.mean(x, axis=0, keepdims=True)
  xc = x - mu
  var = jnp.mean(xc * xc, axis=0, keepdims=True)
  x = xc * lax.rsqrt(var + 1e-5) * g_ref[...] + be_ref[...]
  nrm = jnp.sqrt(jnp.sum(x * x, axis=1, keepdims=True))
  x = x / (nrm + 1e-6)
  o_ref[...] = x
  ob_ref[...] = x.astype(jnp.bfloat16)


def _dense_final_body(h_ref, agg_ref, wt_ref, wb_ref, b_ref, o_ref):
  x = jnp.dot(h_ref[...], wt_ref[...], preferred_element_type=jnp.float32)
  x = x + jnp.dot(agg_ref[...], wb_ref[...],
                  preferred_element_type=jnp.float32)
  o_ref[...] = x + b_ref[...]


_OUT = jax.ShapeDtypeStruct((N, D), jnp.float32)
_OUT_BF = jax.ShapeDtypeStruct((N, D), jnp.bfloat16)
_CP = pltpu.CompilerParams(vmem_limit_bytes=100 * 1024 * 1024)

_dense_bn = pl.pallas_call(_dense_bn_body, out_shape=[_OUT, _OUT_BF],
                           compiler_params=_CP)
_dense_final = pl.pallas_call(_dense_final_body, out_shape=_OUT,
                              compiler_params=_CP)


@jax.jit
def kernel(features, neigh_idx, W0, b0, W1, b1, W2, b2, g0, be0, g1, be1):
  flat = neigh_idx.reshape(-1).astype(jnp.int32)
  flat = jnp.concatenate(
      [flat, jnp.zeros((N_PAD * S - N * S,), jnp.int32)])

  h = features
  hb = features.astype(jnp.bfloat16)
  layers = [(W0, b0, g0, be0), (W1, b1, g1, be1), (W2, b2, None, None)]
  for k, (W, b, g, be) in enumerate(layers):
    agg_i32 = _sc_gather_mean(_pack_bf16(hb), flat)
    agg = _unpack_bf16(agg_i32)[:N]
    wt = W[:D]
    wb = W[D:].astype(jnp.bfloat16)
    b2d = b.reshape(1, D)
    if k < 2:
      h, hb = _dense_bn(h, agg, wt, wb, b2d, g.reshape(1, D),
                        be.reshape(1, D))
    else:
      h = _dense_final(h, agg, wt, wb, b2d)
  return h


# speedup vs baseline: 1.1508x; 1.1508x over previous
import jax
import jax.numpy as jnp
from jax import lax
from jax.experimental import pallas as pl
from jax.experimental.pallas import tpu as pltpu
from jax.experimental.pallas import tpu_sc as plsc

N = 10000
D = 256
S = 25

NUM_CORES = 2
SUBCORES = 16
NODES_PER_WORKER = 320
NODES_PER_CORE = SUBCORES * NODES_PER_WORKER
N_PAD = NUM_CORES * NODES_PER_CORE
GROUP = 8
GROUPS_PER_WORKER = NODES_PER_WORKER // GROUP
IDX_PER_GROUP = GROUP * S
HALF0 = 104
HALF1 = IDX_PER_GROUP - HALF0
LANES = 16
CHUNKS = D // LANES


def _sc_body(h_hbm, idx_hbm, out0_hbm, out1_hbm, idx_all, rows0, rows1,
             out0, out1, semr0, semr1, semo0, semo1):
  core = lax.axis_index("c")
  sub = lax.axis_index("s")
  wid = core * SUBCORES + sub

  idx_base = wid * (NODES_PER_WORKER * S)
  node_base = sub * NODES_PER_WORKER

  rows = (rows0, rows1)
  outs = (out0, out1)
  semr = (semr0, semr1)
  semo = (semo0, semo1)

  pltpu.sync_copy(idx_hbm.at[pl.ds(idx_base, NODES_PER_WORKER * S)], idx_all)

  def issue_gather(g, b):
    off = g * IDX_PER_GROUP
    pltpu.async_copy(h_hbm.at[idx_all.at[pl.ds(off, HALF0)]],
                     rows[b].at[pl.ds(0, HALF0)], semr[b])
    pltpu.async_copy(h_hbm.at[idx_all.at[pl.ds(off + HALF0, HALF1)]],
                     rows[b].at[pl.ds(HALF0, HALF1)], semr[b])

  def wait_gather(b):
    pltpu.make_async_copy(h_hbm.at[pl.ds(0, IDX_PER_GROUP)], rows[b],
                          semr[b]).wait()

  def store(b, gg, dst_hbm):
    pltpu.async_copy(outs[b],
                     dst_hbm.at[pl.ds(node_base + gg * GROUP, GROUP)],
                     semo[b])

  def wait_store(b, dst_hbm):
    pltpu.make_async_copy(outs[b], dst_hbm.at[pl.ds(0, GROUP)],
                          semo[b]).wait()

  issue_gather(0, 0)

  def run(dst_hbm):
    @pl.loop(0, GROUPS_PER_WORKER, step=2)
    def _(g):
      for b in range(2):
        gg = g + b
        nxt = gg + 1

        @pl.when(nxt < GROUPS_PER_WORKER)
        def _():
          issue_gather(nxt, 1 - b)

        wait_gather(b)

        @pl.when(gg >= 2)
        def _():
          wait_store(b, dst_hbm)

        for n in range(GROUP):
          def acc_body(r, accs, n=n):
            row = n * S + r
            return tuple(accs[c] + rows[b][row, pl.ds(c * LANES, LANES)]
                         for c in range(CHUNKS))
          accs = lax.fori_loop(
              0, S, acc_body,
              tuple(jnp.zeros((LANES,), jnp.float32)
                    for _ in range(CHUNKS)),
              unroll=5)
          for c in range(CHUNKS):
            outs[b][n, pl.ds(c * LANES, LANES)] = accs[c] * (1.0 / S)

        store(b, gg, dst_hbm)

    wait_store(0, dst_hbm)
    wait_store(1, dst_hbm)

  @pl.when(core == 0)
  def _():
    run(out0_hbm)

  @pl.when(core == 1)
  def _():
    run(out1_hbm)


@jax.jit
def _sc_gather_mean(h, flat_idx):
  mesh = plsc.VectorSubcoreMesh(core_axis_name="c", subcore_axis_name="s")
  half = jax.ShapeDtypeStruct((NODES_PER_CORE, D), jnp.float32)
  kern = pl.kernel(
      _sc_body,
      out_type=[half, half],
      mesh=mesh,
      scratch_types=[
          pltpu.VMEM((NODES_PER_WORKER * S,), jnp.int32),
          pltpu.VMEM((IDX_PER_GROUP, D), jnp.float32),
          pltpu.VMEM((IDX_PER_GROUP, D), jnp.float32),
          pltpu.VMEM((GROUP, D), jnp.float32),
          pltpu.VMEM((GROUP, D), jnp.float32),
          pltpu.SemaphoreType.DMA,
          pltpu.SemaphoreType.DMA,
          pltpu.SemaphoreType.DMA,
          pltpu.SemaphoreType.DMA,
      ],
  )
  return kern(h, flat_idx)


def _dense_bn_body(h_ref, agg0_ref, agg1_ref, wt_ref, wb_ref, b_ref,
                   g_ref, be_ref, o_ref):
  agg = jnp.concatenate([agg0_ref[...], agg1_ref[...][:N - NODES_PER_CORE]],
                        axis=0)
  x = jnp.dot(h_ref[...], wt_ref[...], preferred_element_type=jnp.float32)
  x = x + jnp.dot(agg, wb_ref[...], preferred_element_type=jnp.float32)
  x = x + b_ref[...]
  x = jnp.maximum(x, 0.0)
  mu = jnp.mean(x, axis=0, keepdims=True)
  xc = x - mu
  var = jnp.mean(xc * xc, axis=0, keepdims=True)
  x = xc * lax.rsqrt(var + 1e-5) * g_ref[...] + be_ref[...]
  nrm = jnp.sqrt(jnp.sum(x * x, axis=1, keepdims=True))
  o_ref[...] = x / (nrm + 1e-6)


def _dense_final_body(h_ref, agg0_ref, agg1_ref, wt_ref, wb_ref, b_ref,
                      o_ref):
  agg = jnp.concatenate([agg0_ref[...], agg1_ref[...][:N - NODES_PER_CORE]],
                        axis=0)
  x = jnp.dot(h_ref[...], wt_ref[...], preferred_element_type=jnp.float32)
  x = x + jnp.dot(agg, wb_ref[...], preferred_element_type=jnp.float32)
  o_ref[...] = x + b_ref[...]


_OUT = jax.ShapeDtypeStruct((N, D), jnp.float32)
_CP = pltpu.CompilerParams(vmem_limit_bytes=100 * 1024 * 1024)

_dense_bn = pl.pallas_call(_dense_bn_body, out_shape=_OUT,
                           compiler_params=_CP)
_dense_final = pl.pallas_call(_dense_final_body, out_shape=_OUT,
                              compiler_params=_CP)


@jax.jit
def kernel(features, neigh_idx, W0, b0, W1, b1, W2, b2, g0, be0, g1, be1):
  flat = neigh_idx.reshape(-1).astype(jnp.int32)
  flat = jnp.concatenate(
      [flat, jnp.zeros((N_PAD * S - N * S,), jnp.int32)])

  h = features
  layers = [(W0, b0, g0, be0), (W1, b1, g1, be1), (W2, b2, None, None)]
  for k, (W, b, g, be) in enumerate(layers):
    agg0, agg1 = _sc_gather_mean(h, flat)
    wt = W[:D]
    wb = W[D:]
    b2d = b.reshape(1, D)
    if k < 2:
      h = _dense_bn(h, agg0, agg1, wt, wb, b2d, g.reshape(1, D),
                    be.reshape(1, D))
    else:
      h = _dense_final(h, agg0, agg1, wt, wb, b2d)
  return h
